# Initial kernel scaffold; baseline (speedup 1.0000x reference)
import jax
import jax.numpy as jnp
from jax.experimental import pallas as pl


def kernel(x, tables):
    raise NotImplementedError("write your pallas kernel here")


# speedup vs baseline: 60.8273x; 60.8273x over previous
import functools

import jax
import jax.numpy as jnp
from jax import lax
from jax.experimental import pallas as pl
from jax.experimental.pallas import tpu as pltpu
from jax.experimental.pallas import tpu_sc as plsc

B = 16384
NUM_NUMERIC = 13
NUM_CAT = 26
VOCAB = 1000
EMB_DIM = 4
ROW_IN = NUM_NUMERIC + NUM_CAT
ROW_OUT = NUM_CAT * EMB_DIM + NUM_NUMERIC
TBL_WORDS = NUM_CAT * VOCAB * EMB_DIM

NW = 32
ROWS_PER_W = B // NW
CHUNK = 128
NCHUNK = ROWS_PER_W // CHUNK
GROUPS = CHUNK // 16


def _sc_embed(x_flat, tbl_flat):
  mesh = plsc.VectorSubcoreMesh(core_axis_name="c", subcore_axis_name="s")

  @functools.partial(
      pl.kernel,
      out_type=jax.ShapeDtypeStruct((B * ROW_OUT,), jnp.float32),
      mesh=mesh,
      scratch_types=[
          pltpu.VMEM((TBL_WORDS,), jnp.float32),
          pltpu.VMEM((CHUNK * ROW_IN,), jnp.float32),
          pltpu.VMEM((CHUNK * ROW_OUT,), jnp.float32),
      ],
      compiler_params=pltpu.CompilerParams(needs_layout_passes=False),
  )
  def k(x_hbm, tbl_hbm, out_hbm, tbl_v, x_v, out_v):
    wid = lax.axis_index("s") * 2 + lax.axis_index("c")
    base = wid * ROWS_PER_W
    pltpu.sync_copy(tbl_hbm, tbl_v)
    iota = lax.iota(jnp.int32, 16)

    def chunk_body(c, carry):
      r0 = base + c * CHUNK
      pltpu.sync_copy(x_hbm.at[pl.ds(r0 * ROW_IN, CHUNK * ROW_IN)], x_v)

      def group_body(g, carry2):
        lrow = iota + g * 16
        xbase = lrow * ROW_IN
        obase = lrow * ROW_OUT
        for f in range(NUM_CAT):
          ids_f = plsc.load_gather(x_v, [xbase + (NUM_NUMERIC + f)])
          idx = ids_f.astype(jnp.int32) * EMB_DIM + f * VOCAB * EMB_DIM
          ob = obase + f * EMB_DIM
          for d in range(EMB_DIM):
            v = plsc.load_gather(tbl_v, [idx + d])
            plsc.store_scatter(out_v, [ob + d], v)
        for j in range(NUM_NUMERIC):
          v = plsc.load_gather(x_v, [xbase + j])
          plsc.store_scatter(out_v, [obase + (NUM_CAT * EMB_DIM + j)], v)
        return carry2

      lax.fori_loop(0, GROUPS, group_body, 0)
      pltpu.sync_copy(out_v, out_hbm.at[pl.ds(r0 * ROW_OUT, CHUNK * ROW_OUT)])
      return carry

    lax.fori_loop(0, NCHUNK, chunk_body, 0)

  return k(x_flat, tbl_flat)


def kernel(x, tables):
  out_flat = _sc_embed(x.reshape(-1), tables.reshape(-1))
  return out_flat.reshape(B, ROW_OUT)


# speedup vs baseline: 118.5853x; 1.9495x over previous
import functools

import jax
import jax.numpy as jnp
from jax import lax
from jax.experimental import pallas as pl
from jax.experimental.pallas import tpu as pltpu
from jax.experimental.pallas import tpu_sc as plsc

B = 16384
NUM_NUMERIC = 13
NUM_CAT = 26
VOCAB = 1000
EMB_DIM = 4
ROW_OUT = NUM_CAT * EMB_DIM + NUM_NUMERIC

GROUPS = B // 16
UNROLL = 8


def _sc_embed(x_cols, tbl_f):
  mesh = plsc.VectorSubcoreMesh(core_axis_name="c", subcore_axis_name="s")

  @functools.partial(
      pl.kernel,
      out_type=jax.ShapeDtypeStruct((ROW_OUT * B,), jnp.float32),
      mesh=mesh,
      scratch_types=[
          pltpu.VMEM((EMB_DIM * VOCAB,), jnp.float32),
          pltpu.VMEM((B,), jnp.float32),
          pltpu.VMEM((B,), jnp.float32),
          pltpu.VMEM((B,), jnp.float32),
          pltpu.VMEM((B,), jnp.float32),
          pltpu.VMEM((B,), jnp.float32),
      ],
      compiler_params=pltpu.CompilerParams(needs_layout_passes=False),
  )
  def k(x_hbm, tbl_hbm, out_hbm, tbl_v, ids_v, e0, e1, e2, e3):
    wid = lax.axis_index("s") * 2 + lax.axis_index("c")
    iota = lax.iota(jnp.int32, 16)
    embs = (e0, e1, e2, e3)

    @pl.when(wid < NUM_CAT)
    def _field_tile():
      f = wid
      pltpu.sync_copy(tbl_hbm.at[pl.ds(f * (EMB_DIM * VOCAB), EMB_DIM * VOCAB)],
                      tbl_v)
      pltpu.sync_copy(x_hbm.at[pl.ds((NUM_NUMERIC + f) * B, B)], ids_v)

      def body(i, carry):
        for u in range(UNROLL):
          pos = (i * UNROLL + u) * 16
          lane = iota + pos
          ids = plsc.load_gather(ids_v, [lane]).astype(jnp.int32)
          for d in range(EMB_DIM):
            v = plsc.load_gather(tbl_v, [ids + d * VOCAB])
            plsc.store_scatter(embs[d], [lane], v)
        return carry

      lax.fori_loop(0, GROUPS // UNROLL, body, 0)
      for d in range(EMB_DIM):
        pltpu.sync_copy(embs[d], out_hbm.at[pl.ds((f * EMB_DIM + d) * B, B)])

    for j in range(NUM_NUMERIC):
      owner = NUM_CAT + (j // 2 if j < 12 else 5)

      @pl.when(wid == owner)
      def _numeric_col(j=j):
        pltpu.sync_copy(x_hbm.at[pl.ds(j * B, B)], ids_v)
        pltpu.sync_copy(ids_v,
                        out_hbm.at[pl.ds((NUM_CAT * EMB_DIM + j) * B, B)])

  return k(x_cols, tbl_f)


def kernel(x, tables):
  x_cols = x.T.reshape(-1)
  tbl_f = tables.transpose(0, 2, 1).reshape(-1)
  out_cols = _sc_embed(x_cols, tbl_f)
  return out_cols.reshape(ROW_OUT, B).T


# speedup vs baseline: 180.6105x; 1.5230x over previous
import functools

import jax
import jax.numpy as jnp
from jax import lax
from jax.experimental import pallas as pl
from jax.experimental.pallas import tpu as pltpu
from jax.experimental.pallas import tpu_sc as plsc

B = 16384
NUM_NUMERIC = 13
NUM_CAT = 26
VOCAB = 1000
EMB_DIM = 4
ROW_OUT = NUM_CAT * EMB_DIM + NUM_NUMERIC

GROUPS = B // 16
UNROLL = 8


def _sc_embed(x_cols, tbl_f):
  mesh = plsc.VectorSubcoreMesh(core_axis_name="c", subcore_axis_name="s")

  @functools.partial(
      pl.kernel,
      out_type=jax.ShapeDtypeStruct((ROW_OUT * B,), jnp.float32),
      mesh=mesh,
      scratch_types=[
          pltpu.VMEM((EMB_DIM * VOCAB,), jnp.float32),
          pltpu.VMEM((B,), jnp.float32),
          pltpu.VMEM((B,), jnp.float32),
          pltpu.VMEM((B,), jnp.float32),
          pltpu.VMEM((B,), jnp.float32),
          pltpu.VMEM((B,), jnp.float32),
      ],
      compiler_params=pltpu.CompilerParams(needs_layout_passes=False),
  )
  def k(x_hbm, tbl_hbm, out_hbm, tbl_v, ids_v, e0, e1, e2, e3):
    wid = lax.axis_index("s") * 2 + lax.axis_index("c")
    iota = lax.iota(jnp.int32, 16)
    embs = (e0, e1, e2, e3)

    @pl.when(wid < NUM_CAT)
    def _field_tile():
      f = wid
      pltpu.sync_copy(tbl_hbm.at[pl.ds(f * (EMB_DIM * VOCAB), EMB_DIM * VOCAB)],
                      tbl_v)
      pltpu.sync_copy(x_hbm.at[pl.ds((NUM_NUMERIC + f) * B, B)], ids_v)

      @plsc.parallel_loop(0, B, step=16, unroll=UNROLL)
      def _body(pos):
        ids = ids_v[pl.ds(pos, 16)].astype(jnp.int32)
        for d in range(EMB_DIM):
          v = plsc.load_gather(tbl_v, [ids + d * VOCAB])
          embs[d][pl.ds(pos, 16)] = v
      for d in range(EMB_DIM):
        pltpu.sync_copy(embs[d], out_hbm.at[pl.ds((f * EMB_DIM + d) * B, B)])

    for j in range(NUM_NUMERIC):
      owner = NUM_CAT + (j // 2 if j < 12 else 5)

      @pl.when(wid == owner)
      def _numeric_col(j=j):
        pltpu.sync_copy(x_hbm.at[pl.ds(j * B, B)], ids_v)
        pltpu.sync_copy(ids_v,
                        out_hbm.at[pl.ds((NUM_CAT * EMB_DIM + j) * B, B)])

  return k(x_cols, tbl_f)


def kernel(x, tables):
  x_cols = x.T.reshape(-1)
  tbl_f = tables.transpose(0, 2, 1).reshape(-1)
  out_cols = _sc_embed(x_cols, tbl_f)
  return out_cols.reshape(ROW_OUT, B).T


# speedup vs baseline: 206.6065x; 1.1439x over previous
import functools

import jax
import jax.numpy as jnp
from jax import lax
from jax.experimental import pallas as pl
from jax.experimental.pallas import tpu as pltpu
from jax.experimental.pallas import tpu_sc as plsc

B = 16384
NUM_NUMERIC = 13
NUM_CAT = 26
VOCAB = 1000
EMB_DIM = 4
ROW_OUT = NUM_CAT * EMB_DIM + NUM_NUMERIC
ROW_PAD = 120
OUT_WORDS = (ROW_PAD // 8) * B * 8

HALF = B // 2
UNROLL = 8


def _sc_embed(x_cols, tbl_f):
  mesh = plsc.VectorSubcoreMesh(core_axis_name="c", subcore_axis_name="s")

  @functools.partial(
      pl.kernel,
      out_type=jax.ShapeDtypeStruct((OUT_WORDS,), jnp.float32),
      mesh=mesh,
      scratch_types=[
          pltpu.VMEM((8 * HALF,), jnp.float32),
          pltpu.VMEM((HALF,), jnp.float32),
          pltpu.VMEM((HALF,), jnp.float32),
          pltpu.VMEM((2 * EMB_DIM * VOCAB,), jnp.float32),
      ],
      compiler_params=pltpu.CompilerParams(needs_layout_passes=False),
  )
  def k(x_hbm, tbl_hbm, out_hbm, emb_v, ids0_v, ids1_v, tbl_v):
    wid = lax.axis_index("s") * 2 + lax.axis_index("c")
    role = wid // 2
    h = wid % 2


    @pl.when(role < NUM_CAT // 2)
    def _field_pair():
      f0 = role * 2
      pltpu.sync_copy(
          tbl_hbm.at[pl.ds(f0 * (EMB_DIM * VOCAB), 2 * EMB_DIM * VOCAB)],
          tbl_v)
      pltpu.sync_copy(
          x_hbm.at[pl.ds((NUM_NUMERIC + f0) * B + h * HALF, HALF)], ids0_v)
      pltpu.sync_copy(
          x_hbm.at[pl.ds((NUM_NUMERIC + f0 + 1) * B + h * HALF, HALF)],
          ids1_v)

      @plsc.parallel_loop(0, HALF, step=16, unroll=UNROLL)
      def _body(pos):
        lb = pos + (pos >> 7) * 896
        ids0 = ids0_v[pl.ds(pos, 16)].astype(jnp.int32)
        for d in range(EMB_DIM):
          v = plsc.load_gather(tbl_v, [ids0 + d * VOCAB])
          emb_v[pl.ds(lb + d * 128, 16)] = v
        ids1 = ids1_v[pl.ds(pos, 16)].astype(jnp.int32)
        for d in range(EMB_DIM):
          v = plsc.load_gather(tbl_v, [ids1 + (EMB_DIM + d) * VOCAB])
          emb_v[pl.ds(lb + (EMB_DIM + d) * 128, 16)] = v

      pltpu.sync_copy(
          emb_v, out_hbm.at[pl.ds((role * 2 + h) * 8 * HALF, 8 * HALF)])

    def _numeric_cols(base_col, cols):
      for j in range(cols):
        pltpu.sync_copy(x_hbm.at[pl.ds((base_col + j) * B + h * HALF, HALF)],
                        ids0_v)

        @plsc.parallel_loop(0, HALF, step=16, unroll=UNROLL)
        def _copy(pos, j=j):
          lb = pos + (pos >> 7) * 896
          emb_v[pl.ds(lb + j * 128, 16)] = ids0_v[pl.ds(pos, 16)]

      pltpu.sync_copy(
          emb_v, out_hbm.at[pl.ds((role * 2 + h) * 8 * HALF, 8 * HALF)])

    @pl.when(role == 13)
    def _numeric_block_a():
      _numeric_cols(0, 8)

    @pl.when(role == 14)
    def _numeric_block_b():
      _numeric_cols(8, 5)

  return k(x_cols, tbl_f)


def kernel(x, tables):
  x_cols = x.T.reshape(-1)
  tbl_f = tables.transpose(0, 2, 1).reshape(-1)
  out_phys = _sc_embed(x_cols, tbl_f)
  out = (out_phys.reshape(ROW_PAD // 8, B // 128, 8, 128)
         .transpose(1, 3, 0, 2)
         .reshape(B, ROW_PAD)[:, :ROW_OUT])
  return out


# speedup vs baseline: 236.1262x; 1.1429x over previous
import functools

import jax
import jax.numpy as jnp
from jax import lax
from jax.experimental import pallas as pl
from jax.experimental.pallas import tpu as pltpu
from jax.experimental.pallas import tpu_sc as plsc

B = 16384
NUM_NUMERIC = 13
NUM_CAT = 26
VOCAB = 1000
EMB_DIM = 4
ROW_OUT = NUM_CAT * EMB_DIM + NUM_NUMERIC
ROW_PAD = 120
OUT_WORDS = (ROW_PAD // 8) * B * 8

HALF = B // 2
QUARTER = B // 4
UNROLL = 8


def _ilv(pos):
  return pos + (pos >> 7) * 896


def _sc_embed(x_cols, tbl_f):
  mesh = plsc.VectorSubcoreMesh(core_axis_name="c", subcore_axis_name="s")

  @functools.partial(
      pl.kernel,
      out_type=jax.ShapeDtypeStruct((OUT_WORDS,), jnp.float32),
      mesh=mesh,
      scratch_types=[
          pltpu.VMEM((8 * HALF,), jnp.float32),
          pltpu.VMEM((HALF,), jnp.float32),
          pltpu.VMEM((HALF,), jnp.float32),
          pltpu.VMEM((2 * EMB_DIM * VOCAB,), jnp.float32),
          pltpu.SemaphoreType.DMA,
          pltpu.SemaphoreType.DMA,
      ],
      compiler_params=pltpu.CompilerParams(needs_layout_passes=False),
  )
  def k(x_hbm, tbl_hbm, out_hbm, emb_v, ids0_v, ids1_v, tbl_v, sem_in,
        sem_out):
    wid = lax.axis_index("s") * 2 + lax.axis_index("c")
    role = wid // 2
    h = wid % 2

    @pl.when(wid < NUM_CAT)
    def _field_pair():
      f0 = role * 2
      d_tbl = pltpu.async_copy(
          tbl_hbm.at[pl.ds(f0 * (EMB_DIM * VOCAB), 2 * EMB_DIM * VOCAB)],
          tbl_v, sem_in)
      d_i0 = pltpu.async_copy(
          x_hbm.at[pl.ds((NUM_NUMERIC + f0) * B + h * HALF, HALF)], ids0_v,
          sem_in)
      d_i1 = pltpu.async_copy(
          x_hbm.at[pl.ds((NUM_NUMERIC + f0 + 1) * B + h * HALF, HALF)],
          ids1_v, sem_in)
      d_tbl.wait()
      d_i0.wait()
      d_i1.wait()
      out_base = (role * 2 + h) * 8 * HALF
      outs = []
      for q in range(2):

        @plsc.parallel_loop(q * (HALF // 2), (q + 1) * (HALF // 2), step=16,
                            unroll=UNROLL)
        def _body(pos):
          lb = _ilv(pos)
          ids0 = ids0_v[pl.ds(pos, 16)].astype(jnp.int32)
          for d in range(EMB_DIM):
            v = plsc.load_gather(tbl_v, [ids0 + d * VOCAB])
            emb_v[pl.ds(lb + d * 128, 16)] = v
          ids1 = ids1_v[pl.ds(pos, 16)].astype(jnp.int32)
          for d in range(EMB_DIM):
            v = plsc.load_gather(tbl_v, [ids1 + (EMB_DIM + d) * VOCAB])
            emb_v[pl.ds(lb + (EMB_DIM + d) * 128, 16)] = v

        outs.append(pltpu.async_copy(
            emb_v.at[pl.ds(q * 4 * HALF, 4 * HALF)],
            out_hbm.at[pl.ds(out_base + q * 4 * HALF, 4 * HALF)], sem_out))
      for d_o in outs:
        d_o.wait()

    def _numeric(base_col, ncols, rbase, nrows, out_off):
      bufs = (ids0_v, ids1_v)
      pend = pltpu.async_copy(
          x_hbm.at[pl.ds(base_col * B + rbase, nrows)],
          bufs[0].at[pl.ds(0, nrows)], sem_in)
      for j in range(ncols):
        nxt = None
        if j + 1 < ncols:
          nxt = pltpu.async_copy(
              x_hbm.at[pl.ds((base_col + j + 1) * B + rbase, nrows)],
              bufs[(j + 1) % 2].at[pl.ds(0, nrows)], sem_in)
        pend.wait()
        pend = nxt
        buf = bufs[j % 2]

        @plsc.parallel_loop(0, nrows, step=16, unroll=UNROLL)
        def _copy(pos, j=j, buf=buf):
          emb_v[pl.ds(_ilv(pos) + j * 128, 16)] = buf[pl.ds(pos, 16)]

      pltpu.sync_copy(emb_v.at[pl.ds(0, 8 * nrows)],
                      out_hbm.at[pl.ds(out_off, 8 * nrows)])

    for q in range(4):

      @pl.when(wid == NUM_CAT + q)
      def _numeric_a(q=q):
        _numeric(0, 8, q * QUARTER, QUARTER,
                 13 * (8 * B) + q * 8 * QUARTER)

    for h2 in range(2):

      @pl.when(wid == NUM_CAT + 4 + h2)
      def _numeric_b(h2=h2):
        _numeric(8, 5, h2 * HALF, HALF, 14 * (8 * B) + h2 * 8 * HALF)

  return k(x_cols, tbl_f)


def kernel(x, tables):
  x_cols = x.T.reshape(-1)
  tbl_f = tables.transpose(0, 2, 1).reshape(-1)
  out_phys = _sc_embed(x_cols, tbl_f)
  out = (out_phys.reshape(ROW_PAD // 8, B // 128, 8, 128)
         .transpose(1, 3, 0, 2)
         .reshape(B, ROW_PAD)[:, :ROW_OUT])
  return out
